# baseline (speedup 1.0000x reference)
import functools

import jax
import jax.numpy as jnp
from jax import lax
from jax.experimental import pallas as pl
from jax.experimental.pallas import tpu as pltpu
from jax.experimental.pallas import tpu_sc as plsc

N_NODES = 50000
N_EDGES = 800000
F_IN = 16
HID = 64
N_GRAPHS = 128
GCN_LAYERS = 3
H_LOCAL = 2
H_GLOBAL = 4
N_CLASSES = 2

NC, NS = 2, 16
NW = NC * NS
NB = 16
BSZ = 3128
NPAD = NB * BSZ
HALF = HID // 2
EROWS = N_EDGES // 128
SUBR = EROWS // 2
RND = 125
NRND = SUBR // RND
SCAP = (RND + 1) * 128
CAPR = 352
CAPE = CAPR * 128
ETAIL = 256
MAXWR = CAPR - RND - 1

_mesh = plsc.VectorSubcoreMesh(
    core_axis_name="c", subcore_axis_name="s", num_cores=NC, num_subcores=NS)
_sc_params = pltpu.CompilerParams(use_tc_tiling_on_sc=False, needs_layout_passes=False)


def _leaky(v):
    return jnp.where(v > 0, v, 0.2 * v)


@functools.partial(
    pl.kernel,
    out_type=[jax.ShapeDtypeStruct((NW * CAPE,), jnp.int32),
              jax.ShapeDtypeStruct((NW * CAPE,), jnp.int32),
              jax.ShapeDtypeStruct((NW * CAPE,), jnp.int32),
              jax.ShapeDtypeStruct((NW * 16,), jnp.int32)],
    mesh=_mesh,
    compiler_params=_sc_params,
    scratch_types=[
        pltpu.VMEM((RND * 128,), jnp.int32),
        pltpu.VMEM((RND * 128,), jnp.int32),
        pltpu.VMEM((SCAP + 16,), jnp.int32),
        pltpu.VMEM((SCAP + 16,), jnp.int32),
        pltpu.VMEM((SCAP + 16,), jnp.int32),
        pltpu.VMEM((16,), jnp.int32),
        pltpu.SemaphoreType.DMA,
    ],
)
def _sc_partition(srcf, dstf, bsrc, bdst, beid, cnts,
                  sbuf, dbuf, stg_s, stg_d, stg_e, cbuf, sem):
    c = lax.axis_index("c")
    s = lax.axis_index("s")
    lo = s * BSZ
    reg = (s * NC + c) * CAPE
    iota = jnp.arange(16, dtype=jnp.int32)

    @pl.loop(0, NRND, init_carry=jnp.int32(0))
    def _round(rnd, wrow):
        ebase = (c * SUBR + rnd * RND) * 128
        pltpu.sync_copy(srcf.at[pl.ds(ebase, RND * 128)], sbuf)
        pltpu.sync_copy(dstf.at[pl.ds(ebase, RND * 128)], dbuf)

        @pl.loop(0, SCAP // 16)
        def _fill(g):
            o = g * 16
            stg_s[pl.ds(o, 16)] = iota + (g % 8) * 16
            stg_d[pl.ds(o, 16)] = jnp.zeros((16,), jnp.int32)
            stg_e[pl.ds(o, 16)] = iota + (g % 16) * 16 + N_EDGES

        @pl.loop(0, RND * 128 // 16, init_carry=jnp.int32(0))
        def _scan(g, w):
            sv = sbuf[pl.ds(g * 16, 16)]
            dv = dbuf[pl.ds(g * 16, 16)] - lo
            m = (dv >= 0) & (dv < BSZ)
            plsc.store_compressed(stg_s.at[pl.ds(w, 16)], sv, mask=m)
            plsc.store_compressed(stg_d.at[pl.ds(w, 16)], dv, mask=m)
            plsc.store_compressed(stg_e.at[pl.ds(w, 16)], ebase + g * 16 + iota, mask=m)
            return w + plsc.all_reduce_population_count(m)[0]

        n = _scan
        nrows = (n + 127) >> 7
        wr = jnp.minimum(wrow, MAXWR)
        pltpu.sync_copy(stg_s.at[pl.ds(0, SCAP)], bsrc.at[pl.ds(reg + wr * 128, SCAP)])
        pltpu.sync_copy(stg_d.at[pl.ds(0, SCAP)], bdst.at[pl.ds(reg + wr * 128, SCAP)])
        pltpu.sync_copy(stg_e.at[pl.ds(0, SCAP)], beid.at[pl.ds(reg + wr * 128, SCAP)])
        return jnp.minimum(wrow + nrows, MAXWR)

    @pl.loop(0, SCAP // 16)
    def _fill2(g):
        o = g * 16
        stg_s[pl.ds(o, 16)] = iota + (g % 8) * 16
        stg_d[pl.ds(o, 16)] = jnp.zeros((16,), jnp.int32)
        stg_e[pl.ds(o, 16)] = iota + (g % 16) * 16 + N_EDGES

    wr2 = jnp.minimum(_round, MAXWR)
    pltpu.sync_copy(stg_s.at[pl.ds(0, SCAP)], bsrc.at[pl.ds(reg + wr2 * 128, SCAP)])
    pltpu.sync_copy(stg_d.at[pl.ds(0, SCAP)], bdst.at[pl.ds(reg + wr2 * 128, SCAP)])
    pltpu.sync_copy(stg_e.at[pl.ds(0, SCAP)], beid.at[pl.ds(reg + wr2 * 128, SCAP)])
    nr8 = jnp.minimum(((_round + 7) >> 3) << 3, MAXWR + 8)
    cbuf[...] = jnp.broadcast_to(nr8, (16,))
    pltpu.sync_copy(cbuf, cnts.at[pl.ds((s * NC + c) * 16, 16)])


@functools.partial(
    pl.kernel,
    out_type=jax.ShapeDtypeStruct((NC * NPAD, HALF), jnp.float32),
    mesh=_mesh,
    compiler_params=_sc_params,
    scratch_types=[
        pltpu.VMEM((512,), jnp.int32),
        pltpu.VMEM((512,), jnp.int32),
        pltpu.VMEM((512,), jnp.int32),
        pltpu.VMEM((512,), jnp.float32),
        pltpu.VMEM((512, HALF), jnp.float32),
        pltpu.VMEM((BSZ, HALF), jnp.float32),
        pltpu.VMEM((16,), jnp.int32),
        pltpu.SemaphoreType.DMA,
        pltpu.SemaphoreType.DMA,
        pltpu.SemaphoreType.DMA,
        pltpu.SemaphoreType.DMA,
        pltpu.SemaphoreType.DMA,
    ],
)
def _sc_propagate(feat0, feat1, bsrc, bdst, beid, cnts, coefx, out,
                  srcb, dstb, eidb, cbuf, rows, accum, ctb, sem, sem2, sem3, sem4, sem5):
    c = lax.axis_index("c")
    s = lax.axis_index("s")
    iota = jnp.arange(16, dtype=jnp.int32)
    zero16 = jnp.zeros((16,), jnp.float32)

    @pl.loop(0, BSZ)
    def _zero(i):
        accum[i, pl.ds(0, 16)] = zero16
        accum[i, pl.ds(16, 16)] = zero16

    for half in range(NC):
        reg = (s * NC + half) * CAPE
        pltpu.sync_copy(cnts.at[pl.ds((s * NC + half) * 16, 16)], ctb)
        nr = ctb[pl.ds(0, 16)][0]

        @pl.loop(0, nr >> 2)
        def _pair(p):
            off = reg + p * 512
            i1 = pltpu.async_copy(bsrc.at[pl.ds(off, 512)], srcb, sem5)
            i2 = pltpu.async_copy(bdst.at[pl.ds(off, 512)], dstb, sem5)
            i3 = pltpu.async_copy(beid.at[pl.ds(off, 512)], eidb, sem5)
            i1.wait()
            i2.wait()
            i3.wait()
            slA = pl.ds(0, 256)
            slB = pl.ds(256, 256)
            ca = pltpu.async_copy(coefx.at[eidb.at[slA]], cbuf.at[slA], sem)
            cb = pltpu.async_copy(coefx.at[eidb.at[slB]], cbuf.at[slB], sem2)

            @pl.when(c == 0)
            def _():
                pltpu.async_copy(feat0.at[srcb.at[slA]], rows.at[slA], sem3)
                pltpu.async_copy(feat0.at[srcb.at[slB]], rows.at[slB], sem4)

            @pl.when(c == 1)
            def _():
                pltpu.async_copy(feat1.at[srcb.at[slA]], rows.at[slA], sem3)
                pltpu.async_copy(feat1.at[srcb.at[slB]], rows.at[slB], sem4)

            def _compute(j0):
                @pl.loop(0, 2)
                def _rowc(j):
                    jj = j + j0
                    for g in range(8):
                        o = jj * 128 + g * 16
                        dl16 = dstb[pl.ds(o, 16)]
                        c16 = cbuf[pl.ds(o, 16)]
                        for i2_ in range(16):
                            r2 = g * 16 + i2_
                            ridx = jnp.broadcast_to(dl16[i2_], (16,))
                            cs = c16[i2_]
                            plsc.addupdate_scatter(accum, [ridx, iota],
                                                   rows[jj * 128 + r2, pl.ds(0, 16)] * cs)
                            plsc.addupdate_scatter(accum, [ridx, iota + 16],
                                                   rows[jj * 128 + r2, pl.ds(16, 16)] * cs)

            ca.wait()
            pltpu.make_async_copy(feat0.at[srcb.at[slA]], rows.at[slA], sem3).wait()
            _compute(0)
            cb.wait()
            pltpu.make_async_copy(feat0.at[srcb.at[slB]], rows.at[slB], sem4).wait()
            _compute(2)

    pltpu.sync_copy(accum, out.at[pl.ds(c * NPAD + s * BSZ, BSZ)])


@functools.partial(
    pl.kernel,
    out_type=[jax.ShapeDtypeStruct((N_EDGES,), jnp.float32),
              jax.ShapeDtypeStruct((NW * NPAD,), jnp.float32)],
    mesh=_mesh,
    compiler_params=_sc_params,
    scratch_types=[
        pltpu.VMEM((512,), jnp.int32),
        pltpu.VMEM((512,), jnp.int32),
        pltpu.VMEM((512,), jnp.float32),
        pltpu.VMEM((512,), jnp.float32),
        pltpu.VMEM((512,), jnp.float32),
        pltpu.VMEM((512,), jnp.float32),
        pltpu.VMEM((NPAD,), jnp.float32),
        pltpu.SemaphoreType.DMA,
        pltpu.SemaphoreType.DMA,
    ],
)
def _sc_att(a_s, a_d, m_t, srcf, dstf, e_out, den_out,
            srcb, dstb, asg, adg, mtg, ebuf, den, sem, sem2):
    c = lax.axis_index("c")
    s = lax.axis_index("s")
    w = s * NC + c
    iota = jnp.arange(16, dtype=jnp.int32)
    lane0 = iota == 0
    zero16 = jnp.zeros((16,), jnp.float32)

    @pl.loop(0, NPAD // 16)
    def _zero(i):
        den[pl.ds(i * 16, 16)] = zero16

    r0 = w * 196
    nr = jnp.minimum(196, EROWS - r0)

    def _block(off, nrows):
        n128 = nrows * 128
        d1 = pltpu.async_copy(srcf.at[pl.ds(off, n128)], srcb.at[pl.ds(0, n128)], sem2)
        d2 = pltpu.async_copy(dstf.at[pl.ds(off, n128)], dstb.at[pl.ds(0, n128)], sem2)
        d1.wait()
        d2.wait()
        gds = []
        for j in range(nrows):
            sl = pl.ds(j * 128, 128)
            gds.append(pltpu.async_copy(a_s.at[srcb.at[sl]], asg.at[sl], sem))
            gds.append(pltpu.async_copy(a_d.at[dstb.at[sl]], adg.at[sl], sem2))
            gds.append(pltpu.async_copy(m_t.at[dstb.at[sl]], mtg.at[sl], sem))
        for d in gds:
            d.wait()

        @pl.loop(0, nrows)
        def _rowc(j):
            for g in range(8):
                sl = pl.ds(j * 128 + g * 16, 16)
                t = asg[sl] + adg[sl]
                l = jnp.where(t > 0, t, 0.2 * t)
                ebuf[sl] = jnp.exp(l - mtg[sl])
            for g in range(8):
                d16 = dstb[pl.ds(j * 128 + g * 16, 16)]
                e16 = ebuf[pl.ds(j * 128 + g * 16, 16)]
                for i2 in range(16):
                    plsc.addupdate_scatter(
                        den, [jnp.broadcast_to(d16[i2], (16,))],
                        jnp.broadcast_to(e16[i2], (16,)), mask=lane0)

        pltpu.sync_copy(ebuf.at[pl.ds(0, n128)], e_out.at[pl.ds(off, n128)])

    @pl.loop(0, nr >> 2)
    def _blk(bi):
        _block((r0 + bi * 4) * 128, 4)

    @pl.when((nr & 3) == 2)
    def _tail():
        _block((r0 + (nr & ~3)) * 128, 2)

    pltpu.sync_copy(den, den_out.at[pl.ds(w * NPAD, NPAD)])


@functools.partial(
    pl.kernel,
    out_type=jax.ShapeDtypeStruct((NW * NPAD,), jnp.float32),
    mesh=_mesh,
    compiler_params=_sc_params,
    scratch_types=[
        pltpu.VMEM((512,), jnp.int32),
        pltpu.VMEM((NPAD,), jnp.float32),
        pltpu.SemaphoreType.DMA,
    ],
)
def _sc_deg(dstf, out, dstb, den, sem):
    c = lax.axis_index("c")
    s = lax.axis_index("s")
    w = s * NC + c
    iota = jnp.arange(16, dtype=jnp.int32)
    lane0 = iota == 0
    zero16 = jnp.zeros((16,), jnp.float32)
    one16 = jnp.ones((16,), jnp.float32)

    @pl.loop(0, NPAD // 16)
    def _zero(i):
        den[pl.ds(i * 16, 16)] = zero16

    r0 = w * 196
    nr = jnp.minimum(196, EROWS - r0)

    def _block(off, nrows):
        pltpu.sync_copy(dstf.at[pl.ds(off, nrows * 128)], dstb.at[pl.ds(0, nrows * 128)])

        @pl.loop(0, nrows * 8)
        def _grp(g):
            d16 = dstb[pl.ds(g * 16, 16)]
            for i2 in range(16):
                plsc.addupdate_scatter(
                    den, [jnp.broadcast_to(d16[i2], (16,))], one16, mask=lane0)

    @pl.loop(0, nr >> 2)
    def _blk(bi):
        _block((r0 + bi * 4) * 128, 4)

    @pl.when((nr & 3) == 2)
    def _tail():
        _block((r0 + (nr & ~3)) * 128, 2)

    pltpu.sync_copy(den, out.at[pl.ds(w * NPAD, NPAD)])


@functools.partial(
    pl.kernel,
    out_type=jax.ShapeDtypeStruct((N_EDGES,), jnp.float32),
    mesh=_mesh,
    compiler_params=_sc_params,
    scratch_types=[
        pltpu.VMEM((128,), jnp.int32),
        pltpu.VMEM((128,), jnp.int32),
        pltpu.VMEM((128,), jnp.float32),
        pltpu.VMEM((128,), jnp.float32),
        pltpu.VMEM((128,), jnp.float32),
        pltpu.SemaphoreType.DMA,
    ],
)
def _sc_norm(dinv, srcf, dstf, out, srcb, dstb, ds_b, dd_b, obuf, sem):
    c = lax.axis_index("c")
    s = lax.axis_index("s")
    w = s * NC + c
    r0 = w * 196
    nr = jnp.minimum(196, EROWS - r0)

    @pl.loop(0, nr)
    def _row(i):
        off = (r0 + i) * 128
        pltpu.sync_copy(srcf.at[pl.ds(off, 128)], srcb)
        pltpu.sync_copy(dstf.at[pl.ds(off, 128)], dstb)
        pltpu.async_copy(dinv.at[srcb], ds_b, sem).wait()
        pltpu.async_copy(dinv.at[dstb], dd_b, sem).wait()
        for g in range(8):
            sl = pl.ds(g * 16, 16)
            obuf[sl] = ds_b[sl] * dd_b[sl]
        pltpu.sync_copy(obuf, out.at[pl.ds(off, 128)])


def _gelu(x):
    return jax.nn.gelu(x, approximate=False)


def _perf(x):
    a1, a2, a3, a4, a5 = 0.254829592, -0.284496736, 1.421413741, -1.453152027, 1.061405429
    s = jnp.sign(x)
    ax = jnp.abs(x)
    t = 1.0 / (1.0 + 0.3275911 * ax)
    poly = ((((a5 * t + a4) * t + a3) * t + a2) * t + a1) * t
    return s * (1.0 - poly * jnp.exp(-ax * ax))


def _pgelu(x):
    return 0.5 * x * (1.0 + _perf(x * 0.7071067811865476))


def _graph_norm(x, batch, w, b, ms, nseg, eps=1e-5):
    cnt = jnp.maximum(jax.ops.segment_sum(jnp.ones((x.shape[0],), x.dtype), batch, nseg), 1.0)[:, None]
    mean = jax.ops.segment_sum(x, batch, nseg) / cnt
    out = x - ms * mean[batch]
    var = jax.ops.segment_sum(out * out, batch, nseg) / cnt
    std = jnp.sqrt(var + eps)
    return w * out / std[batch] + b


def _segment_softmax(s, seg, nseg):
    m = jax.ops.segment_max(s, seg, nseg)
    m = jnp.where(jnp.isfinite(m), m, 0.0)
    e = jnp.exp(s - m[seg])
    d = jax.ops.segment_sum(e, seg, nseg)
    return e / (d[seg] + 1e-16)


def _head_body(hm_ref, hx_ref, ha_ref, c1w, c1b, ln1w, ln1b, c2w, c2b, ln2w,
               ln2b, c3w, c3b, ln3w, ln3b, c4w, c4b, temp, out_ref):
    def ln(v, w, b, eps=1e-5):
        m = v.mean(-1, keepdims=True)
        var = ((v - m) ** 2).mean(-1, keepdims=True)
        return (v - m) / jnp.sqrt(var + eps) * w + b

    h = jnp.concatenate([hm_ref[...], hx_ref[...], ha_ref[...]], axis=1)
    h = _pgelu(ln(h @ c1w[...] + c1b[...], ln1w[...], ln1b[...]))
    h = _pgelu(ln(h @ c2w[...] + c2b[...], ln2w[...], ln2b[...]))
    h = _pgelu(ln(h @ c3w[...] + c3b[...], ln3w[...], ln3b[...]))
    out_ref[...] = (h @ c4w[...] + c4b[...]) / temp[...]


def _head(x_mean, x_max, x_attn, p):
    args = (x_mean, x_max, x_attn,
            p['c1_w'], p['c1_b'][None, :], p['ln1_w'][None, :], p['ln1_b'][None, :],
            p['c2_w'], p['c2_b'][None, :], p['ln2_w'][None, :], p['ln2_b'][None, :],
            p['c3_w'], p['c3_b'][None, :], p['ln3_w'][None, :], p['ln3_b'][None, :],
            p['c4_w'], p['c4_b'][None, :], p['temperature'][None, :])
    return pl.pallas_call(
        _head_body,
        out_shape=jax.ShapeDtypeStruct((N_GRAPHS, N_CLASSES), jnp.float32),
    )(*args)


def _propagate(feat, bkt, coef):
    bsrc, bdst, beid, cnts = bkt
    coefx = jnp.concatenate([coef, jnp.zeros((ETAIL,), jnp.float32)])
    out = _sc_propagate(feat[:, :HALF], feat[:, HALF:], bsrc, bdst, beid, cnts, coefx)
    return jnp.concatenate([out[:N_NODES], out[NC * NPAD - NPAD:NC * NPAD - NPAD + N_NODES]], axis=1)


def _gat(x, srcf, dstf, bkt, w, att_src, att_dst, b, heads):
    n = x.shape[0]
    xl = x @ w
    outs = []
    for h in range(heads):
        xh = xl[:, h * HID:(h + 1) * HID]
        a_s = xh @ att_src[h]
        a_d = xh @ att_dst[h]
        m_t = _leaky(jnp.max(a_s) + a_d)
        e2, den2 = _sc_att(a_s, a_d, m_t, srcf, dstf)
        e_self = jnp.exp(_leaky(a_s + a_d) - m_t)
        den = jnp.sum(den2.reshape(NW, NPAD), axis=0)[:n] + e_self
        num = _propagate(xh, bkt, e2) + e_self[:, None] * xh
        outs.append(num / (den[:, None] + 1e-16))
    return jnp.stack(outs, 0).mean(0) + b


def kernel(x, edge_index, batch, params):
    n = x.shape[0]
    B = N_GRAPHS
    p = params

    srcf = edge_index[0]
    dstf = edge_index[1]
    bkt = _sc_partition(srcf, dstf)

    deg2 = _sc_deg(dstf)
    deg = jnp.sum(deg2.reshape(NW, NPAD), axis=0)[:n] + 1.0
    dinv = lax.rsqrt(deg)
    norm2 = _sc_norm(dinv, srcf, dstf)
    self_coef = (dinv * dinv)[:, None]

    aw = jax.nn.sigmoid(jnp.tanh(x @ p['sa_w1'] + p['sa_b1']) @ p['sa_w2'] + p['sa_b2'])
    x = x * aw
    x = x @ p['in_w'] + p['in_b']
    x = _graph_norm(x, batch, p['in_gn_w'], p['in_gn_b'], p['in_gn_ms'], B)
    x = _gelu(x)
    for i in range(GCN_LAYERS):
        idn = x
        xw = x @ p['gcn%d_w' % i]
        x = _propagate(xw, bkt, norm2) + self_coef * xw + p['gcn%d_b' % i]
        x = _graph_norm(x, batch, p['gcn%d_gn_w' % i], p['gcn%d_gn_b' % i], p['gcn%d_gn_ms' % i], B)
        x = _gelu(x)
        x = x + idn
    idn = x
    xl = _gat(x, srcf, dstf, bkt, p['gl_w'], p['gl_as'], p['gl_ad'], p['gl_b'], H_LOCAL)
    xl = _graph_norm(xl, batch, p['gl_gn_w'], p['gl_gn_b'], p['gl_gn_ms'], B)
    xl = jax.nn.elu(xl)
    xg = _gat(x, srcf, dstf, bkt, p['gg_w'], p['gg_as'], p['gg_ad'], p['gg_b'], H_GLOBAL)
    xg = _graph_norm(xg, batch, p['gg_gn_w'], p['gg_gn_b'], p['gg_gn_ms'], B)
    xg = jax.nn.elu(xg)
    x = (xl + xg) / 2.0 + idn
    cnt = jnp.maximum(jax.ops.segment_sum(jnp.ones((n,), x.dtype), batch, B), 1.0)
    x_mean = jax.ops.segment_sum(x, batch, B) / cnt[:, None]
    x_max = jax.ops.segment_max(x, batch, B)
    x_max = jnp.where(jnp.isfinite(x_max), x_max, 0.0)
    s = jnp.tanh(x @ p['ap_w1'] + p['ap_b1']) @ p['ap_w2'] + p['ap_b2']
    w_att = _segment_softmax(s[:, 0], batch, B)
    x_attn = jax.ops.segment_sum(x * w_att[:, None], batch, B)
    return _head(x_mean, x_max, x_attn, p)


# speedup vs baseline: 1.0523x; 1.0523x over previous
import functools

import jax
import jax.numpy as jnp
from jax import lax
from jax.experimental import pallas as pl
from jax.experimental.pallas import tpu as pltpu
from jax.experimental.pallas import tpu_sc as plsc

N_NODES = 50000
N_EDGES = 800000
F_IN = 16
HID = 64
N_GRAPHS = 128
GCN_LAYERS = 3
H_LOCAL = 2
H_GLOBAL = 4
N_CLASSES = 2

NC, NS = 2, 16
NW = NC * NS
NB = 16
BSZ = 3128
NPAD = NB * BSZ
HALF = HID // 2
EROWS = N_EDGES // 128
SUBR = EROWS // 2
RND = 125
NRND = SUBR // RND
SCAP = (RND + 1) * 128
CAPR = 352
CAPE = CAPR * 128
ETAIL = 256
MAXWR = CAPR - RND - 1

_mesh = plsc.VectorSubcoreMesh(
    core_axis_name="c", subcore_axis_name="s", num_cores=NC, num_subcores=NS)
_sc_params = pltpu.CompilerParams(use_tc_tiling_on_sc=False, needs_layout_passes=False)


def _leaky(v):
    return jnp.where(v > 0, v, 0.2 * v)


@functools.partial(
    pl.kernel,
    out_type=[jax.ShapeDtypeStruct((NW * CAPE,), jnp.int32),
              jax.ShapeDtypeStruct((NW * CAPE,), jnp.int32),
              jax.ShapeDtypeStruct((NW * CAPE,), jnp.int32),
              jax.ShapeDtypeStruct((NW * 16,), jnp.int32)],
    mesh=_mesh,
    compiler_params=_sc_params,
    scratch_types=[
        pltpu.VMEM((RND * 128,), jnp.int32),
        pltpu.VMEM((RND * 128,), jnp.int32),
        pltpu.VMEM((SCAP + 16,), jnp.int32),
        pltpu.VMEM((SCAP + 16,), jnp.int32),
        pltpu.VMEM((SCAP + 16,), jnp.int32),
        pltpu.VMEM((16,), jnp.int32),
        pltpu.SemaphoreType.DMA,
    ],
)
def _sc_partition(srcf, dstf, bsrc, bdst, beid, cnts,
                  sbuf, dbuf, stg_s, stg_d, stg_e, cbuf, sem):
    c = lax.axis_index("c")
    s = lax.axis_index("s")
    lo = s * BSZ
    reg = (s * NC + c) * CAPE
    iota = jnp.arange(16, dtype=jnp.int32)

    @pl.loop(0, NRND, init_carry=jnp.int32(0))
    def _round(rnd, wrow):
        ebase = (c * SUBR + rnd * RND) * 128
        pltpu.sync_copy(srcf.at[pl.ds(ebase, RND * 128)], sbuf)
        pltpu.sync_copy(dstf.at[pl.ds(ebase, RND * 128)], dbuf)

        @pl.loop(0, SCAP // 16)
        def _fill(g):
            o = g * 16
            stg_s[pl.ds(o, 16)] = iota + (g % 8) * 16
            stg_d[pl.ds(o, 16)] = jnp.zeros((16,), jnp.int32)
            stg_e[pl.ds(o, 16)] = iota + (g % 16) * 16 + N_EDGES

        @pl.loop(0, RND * 128 // 16, init_carry=jnp.int32(0))
        def _scan(g, w):
            sv = sbuf[pl.ds(g * 16, 16)]
            dv = dbuf[pl.ds(g * 16, 16)] - lo
            m = (dv >= 0) & (dv < BSZ)
            plsc.store_compressed(stg_s.at[pl.ds(w, 16)], sv, mask=m)
            plsc.store_compressed(stg_d.at[pl.ds(w, 16)], dv, mask=m)
            plsc.store_compressed(stg_e.at[pl.ds(w, 16)], ebase + g * 16 + iota, mask=m)
            return w + plsc.all_reduce_population_count(m)[0]

        n = _scan
        nrows = (n + 127) >> 7
        wr = jnp.minimum(wrow, MAXWR)
        pltpu.sync_copy(stg_s.at[pl.ds(0, SCAP)], bsrc.at[pl.ds(reg + wr * 128, SCAP)])
        pltpu.sync_copy(stg_d.at[pl.ds(0, SCAP)], bdst.at[pl.ds(reg + wr * 128, SCAP)])
        pltpu.sync_copy(stg_e.at[pl.ds(0, SCAP)], beid.at[pl.ds(reg + wr * 128, SCAP)])
        return jnp.minimum(wrow + nrows, MAXWR)

    @pl.loop(0, SCAP // 16)
    def _fill2(g):
        o = g * 16
        stg_s[pl.ds(o, 16)] = iota + (g % 8) * 16
        stg_d[pl.ds(o, 16)] = jnp.zeros((16,), jnp.int32)
        stg_e[pl.ds(o, 16)] = iota + (g % 16) * 16 + N_EDGES

    wr2 = jnp.minimum(_round, MAXWR)
    pltpu.sync_copy(stg_s.at[pl.ds(0, SCAP)], bsrc.at[pl.ds(reg + wr2 * 128, SCAP)])
    pltpu.sync_copy(stg_d.at[pl.ds(0, SCAP)], bdst.at[pl.ds(reg + wr2 * 128, SCAP)])
    pltpu.sync_copy(stg_e.at[pl.ds(0, SCAP)], beid.at[pl.ds(reg + wr2 * 128, SCAP)])
    nr8 = jnp.minimum(((_round + 7) >> 3) << 3, MAXWR + 8)
    cbuf[...] = jnp.broadcast_to(nr8, (16,))
    pltpu.sync_copy(cbuf, cnts.at[pl.ds((s * NC + c) * 16, 16)])


@functools.partial(
    pl.kernel,
    out_type=jax.ShapeDtypeStruct((NC * NPAD, HALF), jnp.float32),
    mesh=_mesh,
    compiler_params=_sc_params,
    scratch_types=[
        pltpu.VMEM((512,), jnp.int32),
        pltpu.VMEM((512,), jnp.int32),
        pltpu.VMEM((512,), jnp.int32),
        pltpu.VMEM((512,), jnp.float32),
        pltpu.VMEM((512, HALF), jnp.float32),
        pltpu.VMEM((BSZ, HALF), jnp.float32),
        pltpu.VMEM((16,), jnp.int32),
        pltpu.SemaphoreType.DMA,
        pltpu.SemaphoreType.DMA,
    ],
)
def _sc_propagate(feat0, feat1, bsrc, bdst, beid, cnts, coefx, out,
                  srcb, dstb, eidb, cbuf, rows, accum, ctb, sem, sem2):
    c = lax.axis_index("c")
    s = lax.axis_index("s")
    iota = jnp.arange(16, dtype=jnp.int32)
    zero16 = jnp.zeros((16,), jnp.float32)

    @pl.loop(0, BSZ)
    def _zero(i):
        accum[i, pl.ds(0, 16)] = zero16
        accum[i, pl.ds(16, 16)] = zero16

    for half in range(NC):
        reg = (s * NC + half) * CAPE
        pltpu.sync_copy(cnts.at[pl.ds((s * NC + half) * 16, 16)], ctb)
        nr = ctb[pl.ds(0, 16)][0]

        @pl.loop(0, nr >> 2)
        def _blk(bi):
            off = reg + bi * 512
            d1 = pltpu.async_copy(bsrc.at[pl.ds(off, 512)], srcb, sem2)
            d2 = pltpu.async_copy(bdst.at[pl.ds(off, 512)], dstb, sem2)
            d3 = pltpu.async_copy(beid.at[pl.ds(off, 512)], eidb, sem2)
            d1.wait()
            d2.wait()
            d3.wait()
            gds = []
            for j in range(4):
                sl = pl.ds(j * 128, 128)
                gds.append(pltpu.async_copy(coefx.at[eidb.at[sl]], cbuf.at[sl], sem))

            @pl.when(c == 0)
            def _():
                fds = [pltpu.async_copy(feat0.at[srcb.at[pl.ds(j * 128, 128)]],
                                        rows.at[pl.ds(j * 128, 128)], sem2)
                       for j in range(4)]
                for d in fds:
                    d.wait()

            @pl.when(c == 1)
            def _():
                fds = [pltpu.async_copy(feat1.at[srcb.at[pl.ds(j * 128, 128)]],
                                        rows.at[pl.ds(j * 128, 128)], sem2)
                       for j in range(4)]
                for d in fds:
                    d.wait()

            for g in gds:
                g.wait()

            @pl.loop(0, 4)
            def _rowc(j):
                for g in range(8):
                    o = j * 128 + g * 16
                    dl16 = dstb[pl.ds(o, 16)]
                    c16 = cbuf[pl.ds(o, 16)]
                    for i2 in range(16):
                        r2 = g * 16 + i2
                        ridx = jnp.broadcast_to(dl16[i2], (16,))
                        cs = c16[i2]
                        plsc.addupdate_scatter(accum, [ridx, iota],
                                               rows[j * 128 + r2, pl.ds(0, 16)] * cs)
                        plsc.addupdate_scatter(accum, [ridx, iota + 16],
                                               rows[j * 128 + r2, pl.ds(16, 16)] * cs)

    pltpu.sync_copy(accum, out.at[pl.ds(c * NPAD + s * BSZ, BSZ)])


@functools.partial(
    pl.kernel,
    out_type=[jax.ShapeDtypeStruct((N_EDGES,), jnp.float32),
              jax.ShapeDtypeStruct((NW * NPAD,), jnp.float32)],
    mesh=_mesh,
    compiler_params=_sc_params,
    scratch_types=[
        pltpu.VMEM((512,), jnp.int32),
        pltpu.VMEM((512,), jnp.int32),
        pltpu.VMEM((512,), jnp.float32),
        pltpu.VMEM((512,), jnp.float32),
        pltpu.VMEM((512,), jnp.float32),
        pltpu.VMEM((512,), jnp.float32),
        pltpu.VMEM((NPAD,), jnp.float32),
        pltpu.SemaphoreType.DMA,
        pltpu.SemaphoreType.DMA,
    ],
)
def _sc_att(a_s, a_d, m_t, srcf, dstf, e_out, den_out,
            srcb, dstb, asg, adg, mtg, ebuf, den, sem, sem2):
    c = lax.axis_index("c")
    s = lax.axis_index("s")
    w = s * NC + c
    iota = jnp.arange(16, dtype=jnp.int32)
    lane0 = iota == 0
    zero16 = jnp.zeros((16,), jnp.float32)

    @pl.loop(0, NPAD // 16)
    def _zero(i):
        den[pl.ds(i * 16, 16)] = zero16

    r0 = w * 196
    nr = jnp.minimum(196, EROWS - r0)

    def _block(off, nrows):
        n128 = nrows * 128
        d1 = pltpu.async_copy(srcf.at[pl.ds(off, n128)], srcb.at[pl.ds(0, n128)], sem2)
        d2 = pltpu.async_copy(dstf.at[pl.ds(off, n128)], dstb.at[pl.ds(0, n128)], sem2)
        d1.wait()
        d2.wait()
        gds = []
        for j in range(nrows):
            sl = pl.ds(j * 128, 128)
            gds.append(pltpu.async_copy(a_s.at[srcb.at[sl]], asg.at[sl], sem))
            gds.append(pltpu.async_copy(a_d.at[dstb.at[sl]], adg.at[sl], sem2))
            gds.append(pltpu.async_copy(m_t.at[dstb.at[sl]], mtg.at[sl], sem))
        for d in gds:
            d.wait()

        @pl.loop(0, nrows)
        def _rowc(j):
            for g in range(8):
                sl = pl.ds(j * 128 + g * 16, 16)
                t = asg[sl] + adg[sl]
                l = jnp.where(t > 0, t, 0.2 * t)
                ebuf[sl] = jnp.exp(l - mtg[sl])
            for g in range(8):
                d16 = dstb[pl.ds(j * 128 + g * 16, 16)]
                e16 = ebuf[pl.ds(j * 128 + g * 16, 16)]
                for i2 in range(16):
                    plsc.addupdate_scatter(
                        den, [jnp.broadcast_to(d16[i2], (16,))],
                        jnp.broadcast_to(e16[i2], (16,)), mask=lane0)

        pltpu.sync_copy(ebuf.at[pl.ds(0, n128)], e_out.at[pl.ds(off, n128)])

    @pl.loop(0, nr >> 2)
    def _blk(bi):
        _block((r0 + bi * 4) * 128, 4)

    @pl.when((nr & 3) == 2)
    def _tail():
        _block((r0 + (nr & ~3)) * 128, 2)

    pltpu.sync_copy(den, den_out.at[pl.ds(w * NPAD, NPAD)])


@functools.partial(
    pl.kernel,
    out_type=jax.ShapeDtypeStruct((NW * NPAD,), jnp.float32),
    mesh=_mesh,
    compiler_params=_sc_params,
    scratch_types=[
        pltpu.VMEM((512,), jnp.int32),
        pltpu.VMEM((NPAD,), jnp.float32),
        pltpu.SemaphoreType.DMA,
    ],
)
def _sc_deg(dstf, out, dstb, den, sem):
    c = lax.axis_index("c")
    s = lax.axis_index("s")
    w = s * NC + c
    iota = jnp.arange(16, dtype=jnp.int32)
    lane0 = iota == 0
    zero16 = jnp.zeros((16,), jnp.float32)
    one16 = jnp.ones((16,), jnp.float32)

    @pl.loop(0, NPAD // 16)
    def _zero(i):
        den[pl.ds(i * 16, 16)] = zero16

    r0 = w * 196
    nr = jnp.minimum(196, EROWS - r0)

    def _block(off, nrows):
        pltpu.sync_copy(dstf.at[pl.ds(off, nrows * 128)], dstb.at[pl.ds(0, nrows * 128)])

        @pl.loop(0, nrows * 8)
        def _grp(g):
            d16 = dstb[pl.ds(g * 16, 16)]
            for i2 in range(16):
                plsc.addupdate_scatter(
                    den, [jnp.broadcast_to(d16[i2], (16,))], one16, mask=lane0)

    @pl.loop(0, nr >> 2)
    def _blk(bi):
        _block((r0 + bi * 4) * 128, 4)

    @pl.when((nr & 3) == 2)
    def _tail():
        _block((r0 + (nr & ~3)) * 128, 2)

    pltpu.sync_copy(den, out.at[pl.ds(w * NPAD, NPAD)])


@functools.partial(
    pl.kernel,
    out_type=jax.ShapeDtypeStruct((N_EDGES,), jnp.float32),
    mesh=_mesh,
    compiler_params=_sc_params,
    scratch_types=[
        pltpu.VMEM((128,), jnp.int32),
        pltpu.VMEM((128,), jnp.int32),
        pltpu.VMEM((128,), jnp.float32),
        pltpu.VMEM((128,), jnp.float32),
        pltpu.VMEM((128,), jnp.float32),
        pltpu.SemaphoreType.DMA,
    ],
)
def _sc_norm(dinv, srcf, dstf, out, srcb, dstb, ds_b, dd_b, obuf, sem):
    c = lax.axis_index("c")
    s = lax.axis_index("s")
    w = s * NC + c
    r0 = w * 196
    nr = jnp.minimum(196, EROWS - r0)

    @pl.loop(0, nr)
    def _row(i):
        off = (r0 + i) * 128
        pltpu.sync_copy(srcf.at[pl.ds(off, 128)], srcb)
        pltpu.sync_copy(dstf.at[pl.ds(off, 128)], dstb)
        pltpu.async_copy(dinv.at[srcb], ds_b, sem).wait()
        pltpu.async_copy(dinv.at[dstb], dd_b, sem).wait()
        for g in range(8):
            sl = pl.ds(g * 16, 16)
            obuf[sl] = ds_b[sl] * dd_b[sl]
        pltpu.sync_copy(obuf, out.at[pl.ds(off, 128)])


def _gelu(x):
    return jax.nn.gelu(x, approximate=False)


def _perf(x):
    a1, a2, a3, a4, a5 = 0.254829592, -0.284496736, 1.421413741, -1.453152027, 1.061405429
    s = jnp.sign(x)
    ax = jnp.abs(x)
    t = 1.0 / (1.0 + 0.3275911 * ax)
    poly = ((((a5 * t + a4) * t + a3) * t + a2) * t + a1) * t
    return s * (1.0 - poly * jnp.exp(-ax * ax))


def _pgelu(x):
    return 0.5 * x * (1.0 + _perf(x * 0.7071067811865476))


def _graph_norm(x, batch, w, b, ms, nseg, eps=1e-5):
    cnt = jnp.maximum(jax.ops.segment_sum(jnp.ones((x.shape[0],), x.dtype), batch, nseg), 1.0)[:, None]
    mean = jax.ops.segment_sum(x, batch, nseg) / cnt
    out = x - ms * mean[batch]
    var = jax.ops.segment_sum(out * out, batch, nseg) / cnt
    std = jnp.sqrt(var + eps)
    return w * out / std[batch] + b


def _segment_softmax(s, seg, nseg):
    m = jax.ops.segment_max(s, seg, nseg)
    m = jnp.where(jnp.isfinite(m), m, 0.0)
    e = jnp.exp(s - m[seg])
    d = jax.ops.segment_sum(e, seg, nseg)
    return e / (d[seg] + 1e-16)


def _head_body(hm_ref, hx_ref, ha_ref, c1w, c1b, ln1w, ln1b, c2w, c2b, ln2w,
               ln2b, c3w, c3b, ln3w, ln3b, c4w, c4b, temp, out_ref):
    def ln(v, w, b, eps=1e-5):
        m = v.mean(-1, keepdims=True)
        var = ((v - m) ** 2).mean(-1, keepdims=True)
        return (v - m) / jnp.sqrt(var + eps) * w + b

    h = jnp.concatenate([hm_ref[...], hx_ref[...], ha_ref[...]], axis=1)
    h = _pgelu(ln(h @ c1w[...] + c1b[...], ln1w[...], ln1b[...]))
    h = _pgelu(ln(h @ c2w[...] + c2b[...], ln2w[...], ln2b[...]))
    h = _pgelu(ln(h @ c3w[...] + c3b[...], ln3w[...], ln3b[...]))
    out_ref[...] = (h @ c4w[...] + c4b[...]) / temp[...]


def _head(x_mean, x_max, x_attn, p):
    args = (x_mean, x_max, x_attn,
            p['c1_w'], p['c1_b'][None, :], p['ln1_w'][None, :], p['ln1_b'][None, :],
            p['c2_w'], p['c2_b'][None, :], p['ln2_w'][None, :], p['ln2_b'][None, :],
            p['c3_w'], p['c3_b'][None, :], p['ln3_w'][None, :], p['ln3_b'][None, :],
            p['c4_w'], p['c4_b'][None, :], p['temperature'][None, :])
    return pl.pallas_call(
        _head_body,
        out_shape=jax.ShapeDtypeStruct((N_GRAPHS, N_CLASSES), jnp.float32),
    )(*args)


def _propagate(feat, bkt, coef):
    bsrc, bdst, beid, cnts = bkt
    coefx = jnp.concatenate([coef, jnp.zeros((ETAIL,), jnp.float32)])
    out = _sc_propagate(feat[:, :HALF], feat[:, HALF:], bsrc, bdst, beid, cnts, coefx)
    return jnp.concatenate([out[:N_NODES], out[NC * NPAD - NPAD:NC * NPAD - NPAD + N_NODES]], axis=1)


def _gat(x, srcf, dstf, bkt, w, att_src, att_dst, b, heads):
    n = x.shape[0]
    xl = x @ w
    outs = []
    for h in range(heads):
        xh = xl[:, h * HID:(h + 1) * HID]
        a_s = xh @ att_src[h]
        a_d = xh @ att_dst[h]
        m_t = _leaky(jnp.max(a_s) + a_d)
        e2, den2 = _sc_att(a_s, a_d, m_t, srcf, dstf)
        e_self = jnp.exp(_leaky(a_s + a_d) - m_t)
        den = jnp.sum(den2.reshape(NW, NPAD), axis=0)[:n] + e_self
        num = _propagate(xh, bkt, e2) + e_self[:, None] * xh
        outs.append(num / (den[:, None] + 1e-16))
    return jnp.stack(outs, 0).mean(0) + b


def kernel(x, edge_index, batch, params):
    n = x.shape[0]
    B = N_GRAPHS
    p = params

    srcf = edge_index[0]
    dstf = edge_index[1]
    bkt = _sc_partition(srcf, dstf)

    deg2 = _sc_deg(dstf)
    deg = jnp.sum(deg2.reshape(NW, NPAD), axis=0)[:n] + 1.0
    dinv = lax.rsqrt(deg)
    norm2 = _sc_norm(dinv, srcf, dstf)
    self_coef = (dinv * dinv)[:, None]

    aw = jax.nn.sigmoid(jnp.tanh(x @ p['sa_w1'] + p['sa_b1']) @ p['sa_w2'] + p['sa_b2'])
    x = x * aw
    x = x @ p['in_w'] + p['in_b']
    x = _graph_norm(x, batch, p['in_gn_w'], p['in_gn_b'], p['in_gn_ms'], B)
    x = _gelu(x)
    for i in range(GCN_LAYERS):
        idn = x
        xw = x @ p['gcn%d_w' % i]
        x = _propagate(xw, bkt, norm2) + self_coef * xw + p['gcn%d_b' % i]
        x = _graph_norm(x, batch, p['gcn%d_gn_w' % i], p['gcn%d_gn_b' % i], p['gcn%d_gn_ms' % i], B)
        x = _gelu(x)
        x = x + idn
    idn = x
    xl = _gat(x, srcf, dstf, bkt, p['gl_w'], p['gl_as'], p['gl_ad'], p['gl_b'], H_LOCAL)
    xl = _graph_norm(xl, batch, p['gl_gn_w'], p['gl_gn_b'], p['gl_gn_ms'], B)
    xl = jax.nn.elu(xl)
    xg = _gat(x, srcf, dstf, bkt, p['gg_w'], p['gg_as'], p['gg_ad'], p['gg_b'], H_GLOBAL)
    xg = _graph_norm(xg, batch, p['gg_gn_w'], p['gg_gn_b'], p['gg_gn_ms'], B)
    xg = jax.nn.elu(xg)
    x = (xl + xg) / 2.0 + idn
    cnt = jnp.maximum(jax.ops.segment_sum(jnp.ones((n,), x.dtype), batch, B), 1.0)
    x_mean = jax.ops.segment_sum(x, batch, B) / cnt[:, None]
    x_max = jax.ops.segment_max(x, batch, B)
    x_max = jnp.where(jnp.isfinite(x_max), x_max, 0.0)
    s = jnp.tanh(x @ p['ap_w1'] + p['ap_b1']) @ p['ap_w2'] + p['ap_b2']
    w_att = _segment_softmax(s[:, 0], batch, B)
    x_attn = jax.ops.segment_sum(x * w_att[:, None], batch, B)
    return _head(x_mean, x_max, x_attn, p)
